# baseline (speedup 1.0000x reference)
import functools

import jax
import jax.numpy as jnp
from jax import lax
from jax.experimental import pallas as pl
from jax.experimental.pallas import tpu as pltpu
from jax.experimental.pallas import tpu_sc as plsc

B = 16384
VOCAB = 1000000
E = 32
IN = 2 * E
BLK = 128
L = 16
NC = 2
NS = 16
NW = NC * NS
BPW = B // NW
CH = 128
NCH = BPW // CH
GPC = CH // L

NPACK = 16384
RPACK = NPACK // 8
PGRID = pl.cdiv(VOCAB, NPACK)
MROWS = PGRID * RPACK
PSH = 14
RSH = 11
HE = E // 2


def _fold_body(w1_ref, b1_ref, w2_ref, b2_ref, w3_ref, b3_ref, w_ref, c_ref):
    w3 = w3_ref[...]
    w23 = jax.lax.dot(w2_ref[...], w3,
                      precision=jax.lax.Precision.HIGHEST)
    w_ref[...] = jax.lax.dot(w1_ref[...], w23,
                             precision=jax.lax.Precision.HIGHEST)
    c_ref[...] = (
        jax.lax.dot(b1_ref[...], w23, precision=jax.lax.Precision.HIGHEST)
        + jax.lax.dot(b2_ref[...], w3, precision=jax.lax.Precision.HIGHEST)
        + b3_ref[...]
    )


_fold = pl.pallas_call(
    _fold_body,
    out_shape=(
        jax.ShapeDtypeStruct((IN, 1), jnp.float32),
        jax.ShapeDtypeStruct((1, 1), jnp.float32),
    ),
)


def _pack_body(t_ref, o_ref):
    tb = t_ref[...].astype(jnp.bfloat16)
    a = lax.bitcast_convert_type(tb[0:HE, :], jnp.uint16).astype(jnp.uint32)
    b = lax.bitcast_convert_type(tb[HE:E, :], jnp.uint16).astype(jnp.uint32)
    merged = lax.bitcast_convert_type(
        jnp.bitwise_or(a, lax.shift_left(b, jnp.uint32(16))), jnp.float32)
    tr = merged.T
    for p in range(8):
        o_ref[:, p * HE:(p + 1) * HE] = tr[p * RPACK:(p + 1) * RPACK, :]


_pack = pl.pallas_call(
    _pack_body,
    grid=(PGRID,),
    in_specs=[pl.BlockSpec((E, NPACK), lambda j: (0, j))],
    out_specs=pl.BlockSpec((RPACK, BLK), lambda j: (j, 0)),
    out_shape=jax.ShapeDtypeStruct((MROWS, BLK), jnp.float32),
)


_mesh = plsc.VectorSubcoreMesh(core_axis_name="c", subcore_axis_name="s",
                               num_cores=NC, num_subcores=NS)


@functools.partial(
    pl.kernel,
    out_type=jax.ShapeDtypeStruct((B,), jnp.float32),
    mesh=_mesh,
    compiler_params=pltpu.CompilerParams(
        needs_layout_passes=False, use_tc_tiling_on_sc=True),
    scratch_types=[
        pltpu.VMEM((NCH, CH), jnp.int32),
        pltpu.VMEM((NCH, CH), jnp.int32),
        pltpu.VMEM((NCH, CH), jnp.int32),
        pltpu.VMEM((NCH, CH), jnp.int32),
        pltpu.VMEM((CH, BLK), jnp.float32),
        pltpu.VMEM((CH, BLK), jnp.float32),
        pltpu.VMEM((CH, BLK), jnp.float32),
        pltpu.VMEM((CH, BLK), jnp.float32),
        pltpu.VMEM((IN * L,), jnp.float32),
        pltpu.VMEM((L,), jnp.float32),
        pltpu.VMEM((BPW,), jnp.float32),
        pltpu.SemaphoreType.DMA,
        pltpu.SemaphoreType.DMA,
        pltpu.SemaphoreType.DMA,
    ],
)
def _sc_main(uf_hbm, if_hbm, ut_hbm, it_hbm, wb_hbm, cb_hbm, out_hbm,
             uidx, iidx, ublk, iblk, ubuf0, ubuf1, ibuf0, ibuf1,
             wv, cv, outv, sem0, sem1, sems):
    wid = lax.axis_index("s") * NC + lax.axis_index("c")
    base = wid * BPW
    ubuf = (ubuf0, ubuf1)
    ibuf = (ibuf0, ibuf1)
    sems_g = (sem0, sem1)

    copies = []
    for k in range(NCH):
        copies.append(pltpu.async_copy(
            uf_hbm.at[pl.ds(base + k * CH, CH)], uidx.at[k], sems))
        copies.append(pltpu.async_copy(
            if_hbm.at[pl.ds(base + k * CH, CH)], iidx.at[k], sems))
    copies.append(pltpu.async_copy(wb_hbm, wv, sems))
    copies.append(pltpu.async_copy(cb_hbm, cv, sems))
    for c in copies:
        c.wait()

    for k in range(NCH):
        for g in range(GPC):
            sl = pl.ds(g * L, L)
            iu = uidx[k, sl]
            ii = iidx[k, sl]
            ublk[k, sl] = jnp.bitwise_or(
                lax.shift_left(lax.shift_right_logical(iu, PSH), RSH),
                jnp.bitwise_and(iu, RPACK - 1))
            iblk[k, sl] = jnp.bitwise_or(
                lax.shift_left(lax.shift_right_logical(ii, PSH), RSH),
                jnp.bitwise_and(ii, RPACK - 1))

    def fire(k):
        b = k % 2
        return (pltpu.async_copy(ut_hbm.at[ublk.at[k]], ubuf[b], sems_g[b]),
                pltpu.async_copy(it_hbm.at[iblk.at[k]], ibuf[b], sems_g[b]))

    lane = lax.iota(jnp.int32, L)
    cvec = cv[...]
    inflight = fire(0)

    for k in range(NCH):
        if k + 1 < NCH:
            nxt = fire(k + 1)
        for c in inflight:
            c.wait()
        b = k % 2
        ub, vb = ubuf[b], ibuf[b]

        def group(g, carry, k=k, ub=ub, vb=vb):
            sl = pl.ds(g * L, L)
            su = lax.shift_left(
                jnp.bitwise_and(lax.shift_right_logical(uidx[k, sl], RSH), 7), 4)
            sv = lax.shift_left(
                jnp.bitwise_and(lax.shift_right_logical(iidx[k, sl], RSH), 7), 4)
            r = lane + g * L
            acc = cvec
            for dd in range(HE):
                gu = lax.bitcast_convert_type(
                    plsc.load_gather(ub, [r, su + dd]), jnp.int32)
                lo = lax.bitcast_convert_type(
                    lax.shift_left(gu, 16), jnp.float32)
                hi = lax.bitcast_convert_type(
                    jnp.bitwise_and(gu, jnp.int32(-65536)), jnp.float32)
                acc = acc + lo * wv[pl.ds(dd * L, L)]
                acc = acc + hi * wv[pl.ds((dd + HE) * L, L)]
                gv = lax.bitcast_convert_type(
                    plsc.load_gather(vb, [r, sv + dd]), jnp.int32)
                lo = lax.bitcast_convert_type(
                    lax.shift_left(gv, 16), jnp.float32)
                hi = lax.bitcast_convert_type(
                    jnp.bitwise_and(gv, jnp.int32(-65536)), jnp.float32)
                acc = acc + lo * wv[pl.ds((E + dd) * L, L)]
                acc = acc + hi * wv[pl.ds((E + dd + HE) * L, L)]
            outv[pl.ds(k * CH + g * L, L)] = 1.0 / (1.0 + jnp.exp(-acc))
            return carry

        lax.fori_loop(0, GPC, group, 0)
        if k + 1 < NCH:
            inflight = nxt

    pltpu.sync_copy(outv, out_hbm.at[pl.ds(base, BPW)])


def kernel(user_feature, item_feature, user_table, item_table,
           W1, b1, W2, b2, W3, b3):
    wf, cf = _fold(W1, b1.reshape(1, -1), W2, b2.reshape(1, -1),
                   W3, b3.reshape(1, 1))
    wbig = jnp.broadcast_to(wf, (IN, L)).reshape(IN * L)
    c16 = jnp.broadcast_to(cf.reshape(1), (L,))
    ut4 = _pack(user_table.T)
    it4 = _pack(item_table.T)
    out = _sc_main(user_feature, item_feature, ut4, it4, wbig, c16)
    return out.reshape(B, 1)


# speedup vs baseline: 1.8146x; 1.8146x over previous
import functools

import jax
import jax.numpy as jnp
from jax import lax
from jax.experimental import pallas as pl
from jax.experimental.pallas import tpu as pltpu
from jax.experimental.pallas import tpu_sc as plsc

B = 16384
VOCAB = 1000000
E = 32
IN = 2 * E
BLK = 128
L = 16
NC = 2
NS = 16
NW = NC * NS
BPW = B // NW
CH = 128
NCH = BPW // CH
GPC = CH // L

NPACK = 16384
RPACK = NPACK // 8
PGRID = pl.cdiv(VOCAB, NPACK)
MROWS = PGRID * RPACK
HE = E // 2


def _fold_body(w1_ref, b1_ref, w2_ref, b2_ref, w3_ref, b3_ref, w_ref, c_ref):
    w3 = w3_ref[...]
    w23 = jax.lax.dot(w2_ref[...], w3,
                      precision=jax.lax.Precision.HIGHEST)
    w_ref[...] = jax.lax.dot(w1_ref[...], w23,
                             precision=jax.lax.Precision.HIGHEST)
    c_ref[...] = (
        jax.lax.dot(b1_ref[...], w23, precision=jax.lax.Precision.HIGHEST)
        + jax.lax.dot(b2_ref[...], w3, precision=jax.lax.Precision.HIGHEST)
        + b3_ref[...]
    )


_fold = pl.pallas_call(
    _fold_body,
    out_shape=(
        jax.ShapeDtypeStruct((IN, 1), jnp.float32),
        jax.ShapeDtypeStruct((1, 1), jnp.float32),
    ),
)


def _pack_body(t_ref, o_ref):
    tb = t_ref[...].astype(jnp.bfloat16)
    a = lax.bitcast_convert_type(
        tb[:, :NPACK // 2], jnp.uint16).astype(jnp.uint32)
    b = lax.bitcast_convert_type(
        tb[:, NPACK // 2:], jnp.uint16).astype(jnp.uint32)
    pr = lax.bitcast_convert_type(
        jnp.bitwise_or(a, lax.shift_left(b, jnp.uint32(16))),
        jnp.float32)
    tr = pr.T
    for p in range(4):
        o_ref[:, p * E:(p + 1) * E] = tr[p * RPACK:(p + 1) * RPACK, :]


_pack = pl.pallas_call(
    _pack_body,
    grid=(PGRID,),
    in_specs=[pl.BlockSpec((E, NPACK), lambda j: (0, j))],
    out_specs=pl.BlockSpec((RPACK, BLK), lambda j: (j, 0)),
    out_shape=jax.ShapeDtypeStruct((MROWS, BLK), jnp.float32),
)


_mesh = plsc.VectorSubcoreMesh(core_axis_name="c", subcore_axis_name="s",
                               num_cores=NC, num_subcores=NS)


@functools.partial(
    pl.kernel,
    out_type=jax.ShapeDtypeStruct((B,), jnp.float32),
    mesh=_mesh,
    compiler_params=pltpu.CompilerParams(
        needs_layout_passes=False, use_tc_tiling_on_sc=True),
    scratch_types=[
        pltpu.VMEM((NCH, CH), jnp.int32),
        pltpu.VMEM((NCH, CH), jnp.int32),
        pltpu.VMEM((NCH, CH), jnp.int32),
        pltpu.VMEM((NCH, CH), jnp.int32),
        pltpu.VMEM((CH, BLK), jnp.float32),
        pltpu.VMEM((CH, BLK), jnp.float32),
        pltpu.VMEM((CH, BLK), jnp.float32),
        pltpu.VMEM((CH, BLK), jnp.float32),
        pltpu.VMEM((IN * L,), jnp.float32),
        pltpu.VMEM((L,), jnp.float32),
        pltpu.VMEM((BPW,), jnp.float32),
        pltpu.SemaphoreType.DMA,
        pltpu.SemaphoreType.DMA,
        pltpu.SemaphoreType.DMA,
    ],
)
def _sc_main(uf_hbm, if_hbm, ut_hbm, it_hbm, wb_hbm, cb_hbm, out_hbm,
             uidx, iidx, ublk, iblk, ubuf0, ubuf1, ibuf0, ibuf1,
             wv, cv, outv, sem0, sem1, sems):
    wid = lax.axis_index("s") * NC + lax.axis_index("c")
    base = wid * BPW
    ubuf = (ubuf0, ubuf1)
    ibuf = (ibuf0, ibuf1)
    sems_g = (sem0, sem1)

    copies = []
    for k in range(NCH):
        copies.append(pltpu.async_copy(
            uf_hbm.at[pl.ds(base + k * CH, CH)], uidx.at[k], sems))
        copies.append(pltpu.async_copy(
            if_hbm.at[pl.ds(base + k * CH, CH)], iidx.at[k], sems))
    copies.append(pltpu.async_copy(wb_hbm, wv, sems))
    copies.append(pltpu.async_copy(cb_hbm, cv, sems))
    for c in copies:
        c.wait()

    for k in range(NCH):
        for g in range(GPC):
            sl = pl.ds(g * L, L)
            iu = uidx[k, sl]
            ii = iidx[k, sl]
            ublk[k, sl] = jnp.bitwise_or(
                lax.shift_left(lax.shift_right_logical(iu, 14), 11),
                jnp.bitwise_and(iu, RPACK - 1))
            iblk[k, sl] = jnp.bitwise_or(
                lax.shift_left(lax.shift_right_logical(ii, 14), 11),
                jnp.bitwise_and(ii, RPACK - 1))

    def fire(k):
        b = k % 2
        return (pltpu.async_copy(ut_hbm.at[ublk.at[k]], ubuf[b], sems_g[b]),
                pltpu.async_copy(it_hbm.at[iblk.at[k]], ibuf[b], sems_g[b]))

    lane = lax.iota(jnp.int32, L)
    cvec = cv[...]
    inflight = fire(0)

    for k in range(NCH):
        if k + 1 < NCH:
            nxt = fire(k + 1)
        for c in inflight:
            c.wait()
        b = k % 2
        ub, vb = ubuf[b], ibuf[b]

        def group(g, carry, k=k, ub=ub, vb=vb):
            sl = pl.ds(g * L, L)
            iu = uidx[k, sl]
            ii = iidx[k, sl]
            su = lax.shift_left(
                jnp.bitwise_and(lax.shift_right_logical(iu, 11), 3), 5)
            sv = lax.shift_left(
                jnp.bitwise_and(lax.shift_right_logical(ii, 11), 3), 5)
            um = jnp.bitwise_and(iu, 8192) == 8192
            vm = jnp.bitwise_and(ii, 8192) == 8192
            r = lane + g * L
            acc = cvec
            for d in range(E):
                gu = lax.bitcast_convert_type(
                    plsc.load_gather(ub, [r, su + d]), jnp.int32)
                ulo = lax.bitcast_convert_type(
                    lax.shift_left(gu, 16), jnp.float32)
                uhi = lax.bitcast_convert_type(
                    jnp.bitwise_and(gu, jnp.int32(-65536)), jnp.float32)
                acc = acc + jnp.where(um, uhi, ulo) * wv[pl.ds(d * L, L)]
                gv = lax.bitcast_convert_type(
                    plsc.load_gather(vb, [r, sv + d]), jnp.int32)
                vlo = lax.bitcast_convert_type(
                    lax.shift_left(gv, 16), jnp.float32)
                vhi = lax.bitcast_convert_type(
                    jnp.bitwise_and(gv, jnp.int32(-65536)), jnp.float32)
                acc = acc + jnp.where(vm, vhi, vlo) * wv[pl.ds((E + d) * L, L)]
            outv[pl.ds(k * CH + g * L, L)] = 1.0 / (1.0 + jnp.exp(-acc))
            return carry

        lax.fori_loop(0, GPC, group, 0)
        if k + 1 < NCH:
            inflight = nxt

    pltpu.sync_copy(outv, out_hbm.at[pl.ds(base, BPW)])


def kernel(user_feature, item_feature, user_table, item_table,
           W1, b1, W2, b2, W3, b3):
    wf, cf = _fold(W1, b1.reshape(1, -1), W2, b2.reshape(1, -1),
                   W3, b3.reshape(1, 1))
    wbig = jnp.broadcast_to(wf, (IN, L)).reshape(IN * L)
    c16 = jnp.broadcast_to(cf.reshape(1), (L,))
    ut4 = _pack(user_table.T)
    it4 = _pack(item_table.T)
    out = _sc_main(user_feature, item_feature, ut4, it4, wbig, c16)
    return out.reshape(B, 1)
